# Initial kernel scaffold; baseline (speedup 1.0000x reference)
import jax
import jax.numpy as jnp
from jax.experimental import pallas as pl


def kernel(feature, segment_ids, W1, b1, W2, b2):
    raise NotImplementedError("write your pallas kernel here")


# speedup vs baseline: 1.7171x; 1.7171x over previous
import jax
import jax.numpy as jnp
from jax.experimental import pallas as pl

NUM_SEGMENTS = 1024
BLOCK = 512


def _tc_body(x_ref, ids_ref, w1_ref, b1_ref, w2_ref, b2_ref, out_ref):
    i = pl.program_id(0)

    @pl.when(i == 0)
    def _init():
        out_ref[...] = jnp.zeros_like(out_ref)

    x = x_ref[...]
    h = jnp.maximum(jnp.dot(x, w1_ref[...], preferred_element_type=jnp.float32) + b1_ref[...], 0.0)
    h = jnp.maximum(jnp.dot(h, w2_ref[...], preferred_element_type=jnp.float32) + b2_ref[...], 0.0)

    ids = ids_ref[...]
    lo = ids_ref[0, 0]
    hi = ids_ref[ids_ref.shape[0] - 1, 0]

    def body(s, carry):
        m = jnp.max(jnp.where(ids == s, h, 0.0), axis=0, keepdims=True)
        out_ref[pl.ds(s, 1), :] = jnp.maximum(out_ref[pl.ds(s, 1), :], m)
        return carry

    jax.lax.fori_loop(lo, hi + 1, body, 0)


def kernel(feature, segment_ids, W1, b1, W2, b2):
    n, d = feature.shape
    ids2 = segment_ids.reshape(n, 1)
    out = pl.pallas_call(
        _tc_body,
        grid=(n // BLOCK,),
        in_specs=[
            pl.BlockSpec((BLOCK, d), lambda i: (i, 0)),
            pl.BlockSpec((BLOCK, 1), lambda i: (i, 0)),
            pl.BlockSpec((d, d), lambda i: (0, 0)),
            pl.BlockSpec((1, d), lambda i: (0, 0)),
            pl.BlockSpec((d, d), lambda i: (0, 0)),
            pl.BlockSpec((1, d), lambda i: (0, 0)),
        ],
        out_specs=pl.BlockSpec((NUM_SEGMENTS, d), lambda i: (0, 0)),
        out_shape=jax.ShapeDtypeStruct((NUM_SEGMENTS, d), jnp.float32),
    )(feature, ids2, W1, b1.reshape(1, d), W2, b2.reshape(1, d))
    return out
